# Initial kernel scaffold; baseline (speedup 1.0000x reference)
import jax
import jax.numpy as jnp
from jax.experimental import pallas as pl


def kernel(x, edge_index, W1, b1, W2, b2):
    raise NotImplementedError("write your pallas kernel here")


# speedup vs baseline: 33.0822x; 33.0822x over previous
import functools
import jax
import jax.numpy as jnp
from jax import lax
from jax.experimental import pallas as pl
from jax.experimental.pallas import tpu as pltpu
from jax.experimental.pallas import tpu_sc as plsc

N = 10000
E = 320000
D = 128
H = 128

NC = 2
NS = 16
NW = NC * NS
L = 16

NP = 10240
EW = E // NW
CK = 80
CH = EW // CK
RPT = NP // NS

_mesh = plsc.VectorSubcoreMesh(
    core_axis_name="c", subcore_axis_name="s", num_cores=NC, num_subcores=NS
)
_sc_params = pltpu.CompilerParams(needs_layout_passes=False)


@functools.partial(
    pl.kernel,
    out_type=jax.ShapeDtypeStruct((NW, NP), jnp.float32),
    mesh=_mesh,
    compiler_params=_sc_params,
    scratch_types=[
        pltpu.VMEM((EW,), jnp.int32),
        pltpu.VMEM((NP,), jnp.float32),
    ],
)
def _deg_kernel(dst_hbm, out_hbm, idx_v, acc_v):
    c = lax.axis_index("c")
    s = lax.axis_index("s")
    w = s * NC + c

    pltpu.sync_copy(dst_hbm.at[pl.ds(w * EW, EW)], idx_v)

    zeros = jnp.zeros((L,), jnp.float32)

    def zero_body(i, _):
        acc_v[pl.ds(i * L, L)] = zeros
        return 0

    lax.fori_loop(0, NP // L, zero_body, 0)

    ones = jnp.ones((L,), jnp.float32)

    def body(i, _):
        idx = idx_v[pl.ds(i * L, L)]
        plsc.addupdate_scatter(acc_v, [idx], ones)
        return 0

    lax.fori_loop(0, EW // L, body, 0)
    pltpu.sync_copy(acc_v, out_hbm.at[w])


@functools.partial(
    pl.kernel,
    out_type=jax.ShapeDtypeStruct((NC, NP, H), jnp.float32),
    mesh=_mesh,
    compiler_params=_sc_params,
    scratch_types=[
        pltpu.VMEM((CH, CK), jnp.int32),
        pltpu.VMEM((CH, CK), jnp.int32),
        pltpu.VMEM((CK, H), jnp.float32),
        pltpu.VMEM_SHARED((NP, H), jnp.float32),
    ],
)
def _row_agg_kernel(src_hbm, dst_hbm, p_hbm, out_hbm, sidx_v, didx_v, rows_v, acc_sh):
    c = lax.axis_index("c")
    s = lax.axis_index("s")
    w = s * NC + c

    pltpu.sync_copy(src_hbm.at[w], sidx_v)
    pltpu.sync_copy(dst_hbm.at[w], didx_v)

    zeros = jnp.zeros((L,), jnp.float32)

    def zrow(i, _):
        r = i // (H // L)
        col = (i % (H // L)) * L
        rows_v[r, pl.ds(col, L)] = zeros
        return 0

    lax.fori_loop(0, CK * (H // L), zrow, 0)

    def zcopy(j, _):
        pltpu.sync_copy(rows_v, acc_sh.at[pl.ds(s * RPT + j * CK, CK)])
        return 0

    lax.fori_loop(0, RPT // CK, zcopy, 0)
    plsc.subcore_barrier()

    def body(i, _):
        pltpu.sync_copy(p_hbm.at[sidx_v.at[i]], rows_v)
        pltpu.sync_copy(rows_v, acc_sh.at[didx_v.at[i]], add=True)
        return 0

    lax.fori_loop(0, CH, body, 0)
    plsc.subcore_barrier()

    pltpu.sync_copy(
        acc_sh.at[pl.ds(s * RPT, RPT)], out_hbm.at[c, pl.ds(s * RPT, RPT)]
    )


@functools.partial(
    pl.kernel,
    out_type=jax.ShapeDtypeStruct((NW, NP), jnp.float32),
    mesh=_mesh,
    compiler_params=_sc_params,
    scratch_types=[
        pltpu.VMEM((EW,), jnp.int32),
        pltpu.VMEM((EW,), jnp.int32),
        pltpu.VMEM((NP,), jnp.float32),
        pltpu.VMEM((NP,), jnp.float32),
    ],
)
def _scalar_agg_kernel(src_hbm, dst_hbm, q_hbm, out_hbm, sidx_v, didx_v, q_v, acc_v):
    c = lax.axis_index("c")
    s = lax.axis_index("s")
    w = s * NC + c

    pltpu.sync_copy(src_hbm.at[pl.ds(w * EW, EW)], sidx_v)
    pltpu.sync_copy(dst_hbm.at[pl.ds(w * EW, EW)], didx_v)
    pltpu.sync_copy(q_hbm, q_v)

    zeros = jnp.zeros((L,), jnp.float32)

    def zero_body(i, _):
        acc_v[pl.ds(i * L, L)] = zeros
        return 0

    lax.fori_loop(0, NP // L, zero_body, 0)

    def body(i, _):
        sv = sidx_v[pl.ds(i * L, L)]
        val = plsc.load_gather(q_v, [sv])
        dv = didx_v[pl.ds(i * L, L)]
        plsc.addupdate_scatter(acc_v, [dv], val)
        return 0

    lax.fori_loop(0, EW // L, body, 0)
    pltpu.sync_copy(acc_v, out_hbm.at[w])


_RB = 2560


def _k2_body(degp_ref, x_ref, w1_ref, p_ref, dinv_ref):
    deg = jnp.sum(degp_ref[...], axis=1, keepdims=True) + 1.0
    dinv = lax.rsqrt(deg)
    h = jnp.dot(x_ref[...], w1_ref[...], preferred_element_type=jnp.float32)
    p_ref[...] = h * dinv
    dinv_ref[...] = dinv


def _k4_body(agg0_ref, agg1_ref, p_ref, dinv_ref, b1_ref, w2_ref, qp_ref):
    dinv = dinv_ref[...]
    out1 = dinv * (agg0_ref[...] + agg1_ref[...] + p_ref[...]) + b1_ref[...]
    h2 = jnp.maximum(out1, 0.0)
    q = jnp.dot(h2, w2_ref[...], preferred_element_type=jnp.float32)
    qp_ref[...] = q * dinv


def _k6_body(aggp_ref, qp_ref, dinv_ref, b2_ref, out_ref):
    agg2 = jnp.sum(aggp_ref[...], axis=1, keepdims=True)
    out_ref[...] = dinv_ref[...] * (agg2 + qp_ref[...]) + b2_ref[...]


def kernel(x, edge_index, W1, b1, W2, b2):
    edge_index = edge_index.astype(jnp.int32)
    src = edge_index[0]
    dst = edge_index[1]
    src3 = src.reshape(NW, CH, CK)
    dst3 = dst.reshape(NW, CH, CK)

    x_pad = jnp.zeros((NP, D), jnp.float32).at[:N].set(x)

    degp = _deg_kernel(dst)
    degp_t = degp.T

    grid = NP // _RB
    p, dinv = pl.pallas_call(
        _k2_body,
        grid=(grid,),
        in_specs=[
            pl.BlockSpec((_RB, NW), lambda i: (i, 0)),
            pl.BlockSpec((_RB, D), lambda i: (i, 0)),
            pl.BlockSpec((D, H), lambda i: (0, 0)),
        ],
        out_specs=[
            pl.BlockSpec((_RB, H), lambda i: (i, 0)),
            pl.BlockSpec((_RB, 1), lambda i: (i, 0)),
        ],
        out_shape=[
            jax.ShapeDtypeStruct((NP, H), jnp.float32),
            jax.ShapeDtypeStruct((NP, 1), jnp.float32),
        ],
    )(degp_t, x_pad, W1)

    aggp = _row_agg_kernel(src3, dst3, p)

    qp = pl.pallas_call(
        _k4_body,
        grid=(grid,),
        in_specs=[
            pl.BlockSpec((_RB, H), lambda i: (i, 0)),
            pl.BlockSpec((_RB, H), lambda i: (i, 0)),
            pl.BlockSpec((_RB, H), lambda i: (i, 0)),
            pl.BlockSpec((_RB, 1), lambda i: (i, 0)),
            pl.BlockSpec((1, H), lambda i: (0, 0)),
            pl.BlockSpec((H, 1), lambda i: (0, 0)),
        ],
        out_specs=pl.BlockSpec((_RB, 1), lambda i: (i, 0)),
        out_shape=jax.ShapeDtypeStruct((NP, 1), jnp.float32),
    )(aggp[0], aggp[1], p, dinv, b1.reshape(1, H), W2)

    agg2p = _scalar_agg_kernel(src, dst, qp.reshape(NP))
    agg2p_t = agg2p.T

    out = pl.pallas_call(
        _k6_body,
        grid=(grid,),
        in_specs=[
            pl.BlockSpec((_RB, NW), lambda i: (i, 0)),
            pl.BlockSpec((_RB, 1), lambda i: (i, 0)),
            pl.BlockSpec((_RB, 1), lambda i: (i, 0)),
            pl.BlockSpec((1, 1), lambda i: (0, 0)),
        ],
        out_specs=pl.BlockSpec((_RB, 1), lambda i: (i, 0)),
        out_shape=jax.ShapeDtypeStruct((NP, 1), jnp.float32),
    )(agg2p_t, qp, dinv, b2.reshape(1, 1))

    return out[:N, 0]
